# baseline (speedup 1.0000x reference)
import jax
import jax.numpy as jnp
from jax import lax
from jax.experimental import pallas as pl
from jax.experimental.pallas import tpu as pltpu
from jax.experimental.pallas import tpu_sc as plsc

N = 10000
NA = 10112
E = 320000
NC = 2
NS = 16
CH = 128
K = 81
TOT = NC * NS * K
KD = 88
DTOT = NC * NS * KD
EP = DTOT * CH
STRIPE = NA // NS
_MESH = plsc.VectorSubcoreMesh(core_axis_name="c", subcore_axis_name="s")
_F32 = jnp.float32


_NB = 3


def _agg_body(tab_hbm, srcs_hbm, dsts_hbm, z_hbm, out_hbm,
              x0, x1, x2, d0, d1, d2, b0, b1, b2, acc,
              i0, i1, i2, e0, e1, e2, g0, g1, g2, s0, s1, s2):
    sidx = (x0, x1, x2)
    didx = (d0, d1, d2)
    bufs = (b0, b1, b2)
    isems = (i0, i1, i2)
    dsems = (e0, e1, e2)
    gsems = (g0, g1, g2)
    ssems = (s0, s1, s2)
    c = lax.axis_index("c")
    s = lax.axis_index("s")
    base = (c * NS + s) * K
    pltpu.sync_copy(z_hbm, acc.at[pl.ds(s * STRIPE, STRIPE)])
    plsc.subcore_barrier()

    for i in range(_NB):
        pltpu.async_copy(srcs_hbm.at[pl.ds(base + i, 1)], sidx[i], isems[i])
        pltpu.async_copy(dsts_hbm.at[pl.ds(base + i, 1)], didx[i], dsems[i])
    for i in range(_NB):
        pltpu.make_async_copy(srcs_hbm.at[pl.ds(base + i, 1)],
                              sidx[i], isems[i]).wait()
        pltpu.async_copy(tab_hbm.at[sidx[i].at[0]], bufs[i], gsems[i])

    def body(t, carry):
        j = _NB * t
        for i in range(_NB):
            pltpu.make_async_copy(tab_hbm.at[sidx[i].at[0]],
                                  bufs[i], gsems[i]).wait()
            pltpu.make_async_copy(dsts_hbm.at[pl.ds(base + j + i, 1)],
                                  didx[i], dsems[i]).wait()
            pltpu.async_copy(bufs[i], acc.at[didx[i].at[0]], ssems[i],
                             add=True)

            @pl.when(j + i + _NB < K)
            def _():
                pltpu.async_copy(srcs_hbm.at[pl.ds(base + j + i + _NB, 1)],
                                 sidx[i], isems[i])
        for i in range(_NB):
            pltpu.make_async_copy(bufs[i], acc.at[didx[i].at[0]],
                                  ssems[i]).wait()

            @pl.when(j + i + _NB < K)
            def _():
                pltpu.async_copy(dsts_hbm.at[pl.ds(base + j + i + _NB, 1)],
                                 didx[i], dsems[i])
                pltpu.make_async_copy(srcs_hbm.at[pl.ds(base + j + i + _NB, 1)],
                                      sidx[i], isems[i]).wait()
                pltpu.async_copy(tab_hbm.at[sidx[i].at[0]], bufs[i], gsems[i])
        return carry

    lax.fori_loop(0, K // _NB, body, 0)
    plsc.subcore_barrier()
    pltpu.sync_copy(acc.at[pl.ds(s * STRIPE, STRIPE)],
                    out_hbm.at[c, pl.ds(s * STRIPE, STRIPE)])


_agg = pl.kernel(
    _agg_body,
    out_type=jax.ShapeDtypeStruct((NC, NA, 128), _F32),
    mesh=_MESH,
    scratch_types=[
        pltpu.VMEM((1, CH), jnp.int32),
        pltpu.VMEM((1, CH), jnp.int32),
        pltpu.VMEM((1, CH), jnp.int32),
        pltpu.VMEM((1, CH), jnp.int32),
        pltpu.VMEM((1, CH), jnp.int32),
        pltpu.VMEM((1, CH), jnp.int32),
        pltpu.VMEM((CH, 128), _F32),
        pltpu.VMEM((CH, 128), _F32),
        pltpu.VMEM((CH, 128), _F32),
        pltpu.VMEM_SHARED((NA, 128), _F32),
        pltpu.SemaphoreType.DMA,
        pltpu.SemaphoreType.DMA,
        pltpu.SemaphoreType.DMA,
        pltpu.SemaphoreType.DMA,
        pltpu.SemaphoreType.DMA,
        pltpu.SemaphoreType.DMA,
        pltpu.SemaphoreType.DMA,
        pltpu.SemaphoreType.DMA,
        pltpu.SemaphoreType.DMA,
        pltpu.SemaphoreType.DMA,
        pltpu.SemaphoreType.DMA,
        pltpu.SemaphoreType.DMA,
    ],
)


def _deg_body(dsts_hbm, ones_hbm, z_hbm, out_hbm, dst_v, ones_v, acc):
    c = lax.axis_index("c")
    s = lax.axis_index("s")
    wid = s * NC + c
    pltpu.sync_copy(z_hbm, acc.at[pl.ds(s * STRIPE, STRIPE)])
    pltpu.sync_copy(ones_hbm, ones_v)
    pltpu.sync_copy(dsts_hbm.at[pl.ds(wid * KD, KD)], dst_v)
    plsc.subcore_barrier()

    def chunk(j, carry):
        pltpu.sync_copy(ones_v, acc.at[dst_v.at[j]], add=True)
        return carry

    lax.fori_loop(0, KD, chunk, 0)
    plsc.subcore_barrier()
    pltpu.sync_copy(acc.at[pl.ds(s * STRIPE, STRIPE)],
                    out_hbm.at[c, pl.ds(s * STRIPE, STRIPE)])


_deg = pl.kernel(
    _deg_body,
    out_type=jax.ShapeDtypeStruct((NC, NA, 128), _F32),
    mesh=_MESH,
    scratch_types=[
        pltpu.VMEM((KD, CH), jnp.int32),
        pltpu.VMEM((CH, 128), _F32),
        pltpu.VMEM_SHARED((NA, 128), _F32),
    ],
)


_R = 1000


def _tc1_body(degp, x, dinv_b, xs):
    deg = degp[0] + degp[1] + 1.0
    db = lax.rsqrt(deg)
    dinv_b[...] = db
    xs[...] = x[...] * db


def _tc2_body(p1, x, dinv, w1, b1, h1, hs1a, hs1b):
    db = dinv[...]
    z = db * (p1[0] + p1[1]) + db * db * x[...]
    h = jnp.maximum(
        jnp.dot(z, w1[...], preferred_element_type=_F32,
                precision=lax.Precision.HIGHEST) + b1[...], 0.0)
    h1[...] = h
    d256 = jnp.broadcast_to(db[:, 0:1], (_R, 256))
    hs = h * d256
    hs1a[...] = hs[:, :128]
    hs1b[...] = hs[:, 128:]


def _tc3_body(p2a, p2b, h1, dinv, w2, b2, w3, g, gs):
    db = dinv[...]
    d256 = jnp.broadcast_to(db[:, 0:1], (_R, 256))
    agg = jnp.concatenate([p2a[0] + p2a[1], p2b[0] + p2b[1]], axis=1)
    z2 = d256 * agg + d256 * d256 * h1[...]
    h2 = jnp.maximum(
        jnp.dot(z2, w2[...], preferred_element_type=_F32,
                precision=lax.Precision.HIGHEST) + b2[...], 0.0)
    gg = jnp.dot(h2, w3[...], preferred_element_type=_F32,
                 precision=lax.Precision.HIGHEST)
    g[...] = gg
    gs[...] = gg * db


def _tc4_body(p3, g, dinv, b3, out):
    db = dinv[...]
    out[...] = db * (p3[0] + p3[1]) + db * db * g[...] + b3[...]


def _row_spec(d):
    return pl.BlockSpec((_R, d), lambda i: (i, 0))


def _part_spec(d):
    return pl.BlockSpec((NC, _R, d), lambda i: (0, i, 0))


def _full_spec(a, b):
    return pl.BlockSpec((a, b), lambda i: (0, 0))


_GRID = N // _R

_tc1 = pl.pallas_call(
    _tc1_body,
    grid=(_GRID,),
    in_specs=[_part_spec(128), _row_spec(128)],
    out_specs=[_row_spec(128), _row_spec(128)],
    out_shape=[jax.ShapeDtypeStruct((N, 128), _F32)] * 2,
)

_tc2 = pl.pallas_call(
    _tc2_body,
    grid=(_GRID,),
    in_specs=[_part_spec(128), _row_spec(128), _row_spec(128),
              _full_spec(128, 256), _full_spec(1, 256)],
    out_specs=[_row_spec(256), _row_spec(128), _row_spec(128)],
    out_shape=[jax.ShapeDtypeStruct((N, 256), _F32),
               jax.ShapeDtypeStruct((N, 128), _F32),
               jax.ShapeDtypeStruct((N, 128), _F32)],
)

_tc3 = pl.pallas_call(
    _tc3_body,
    grid=(_GRID,),
    in_specs=[_part_spec(128), _part_spec(128), _row_spec(256), _row_spec(128),
              _full_spec(256, 256), _full_spec(1, 256), _full_spec(256, 128)],
    out_specs=[_row_spec(128), _row_spec(128)],
    out_shape=[jax.ShapeDtypeStruct((N, 128), _F32)] * 2,
)

_tc4 = pl.pallas_call(
    _tc4_body,
    grid=(_GRID,),
    in_specs=[_part_spec(128), _row_spec(128), _row_spec(128),
              _full_spec(1, 128)],
    out_specs=_row_spec(128),
    out_shape=jax.ShapeDtypeStruct((N, 128), _F32),
)


@jax.jit
def kernel(x, edge_index, W1, b1, W2, b2, W3, b3):
    pad = EP - E
    pad_src = (jnp.arange(pad, dtype=jnp.int32) % 128)
    pad_dst = N + (jnp.arange(pad, dtype=jnp.int32) % (NA - N))
    src = jnp.concatenate([edge_index[0], pad_src]).reshape(DTOT, CH)
    dst = jnp.concatenate([edge_index[1], pad_dst]).reshape(DTOT, CH)
    z128 = jnp.zeros((STRIPE, 128), _F32)
    ones128 = jnp.ones((CH, 128), _F32)

    degp = _deg(dst, ones128, z128)
    dinv_b, xs = _tc1(degp, x)
    p1 = _agg(xs, src, dst, z128)
    h1, hs1a, hs1b = _tc2(p1, x, dinv_b, W1, b1.reshape(1, 256))
    p2a = _agg(hs1a, src, dst, z128)
    p2b = _agg(hs1b, src, dst, z128)
    g, gs = _tc3(p2a, p2b, h1, dinv_b, W2, b2.reshape(1, 256), W3)
    p3 = _agg(gs, src, dst, z128)
    return _tc4(p3, g, dinv_b, b3.reshape(1, 128))


# speedup vs baseline: 1.1246x; 1.1246x over previous
import jax
import jax.numpy as jnp
from jax import lax
from jax.experimental import pallas as pl
from jax.experimental.pallas import tpu as pltpu
from jax.experimental.pallas import tpu_sc as plsc

N = 10000
NA = 10112
E = 320000
NC = 2
NS = 16
CH = 128
K1 = 80
K0 = 80
TOT = NS * (K0 + K1)
EP = TOT * CH
KD = TOT // (NC * NS)
STRIPE = NA // NS
_MESH = plsc.VectorSubcoreMesh(core_axis_name="c", subcore_axis_name="s")
_F32 = jnp.float32


def _agg_body(tab_hbm, srcs_hbm, dsts_hbm, z_hbm, out_hbm,
              dst_v, x0, x1, b0, b1, acc,
              i0, i1, g0, g1):
    sidx = (x0, x1)
    bufs = (b0, b1)
    isems = (i0, i1)
    gsems = (g0, g1)
    c = lax.axis_index("c")
    s = lax.axis_index("s")
    base = jnp.where(c == 1, s * K1, NS * K1 + s * K0)
    kc = jnp.where(c == 1, K1, K0)
    pltpu.sync_copy(z_hbm, acc.at[pl.ds(s * STRIPE, STRIPE)])
    pltpu.sync_copy(dsts_hbm.at[pl.ds(base, K0)], dst_v)
    plsc.subcore_barrier()

    for i in range(2):
        pltpu.async_copy(srcs_hbm.at[pl.ds(base + i, 1)], sidx[i], isems[i])
    for i in range(2):
        pltpu.make_async_copy(srcs_hbm.at[pl.ds(base + i, 1)],
                              sidx[i], isems[i]).wait()
        pltpu.async_copy(tab_hbm.at[sidx[i].at[0]], bufs[i], gsems[i])

    def body(t, carry):
        j = 2 * t
        for i in range(2):
            pltpu.make_async_copy(tab_hbm.at[sidx[i].at[0]],
                                  bufs[i], gsems[i]).wait()

            @pl.when(j + i + 2 < kc)
            def _():
                pltpu.async_copy(srcs_hbm.at[pl.ds(base + j + i + 2, 1)],
                                 sidx[i], isems[i])

            pltpu.sync_copy(bufs[i], acc.at[dst_v.at[j + i]], add=True)

            @pl.when(j + i + 2 < kc)
            def _():
                pltpu.make_async_copy(srcs_hbm.at[pl.ds(base + j + i + 2, 1)],
                                      sidx[i], isems[i]).wait()
                pltpu.async_copy(tab_hbm.at[sidx[i].at[0]], bufs[i], gsems[i])
        return carry

    lax.fori_loop(0, kc // 2, body, 0)
    plsc.subcore_barrier()
    pltpu.sync_copy(acc.at[pl.ds(s * STRIPE, STRIPE)],
                    out_hbm.at[c, pl.ds(s * STRIPE, STRIPE)])


_agg = pl.kernel(
    _agg_body,
    out_type=jax.ShapeDtypeStruct((NC, NA, 128), _F32),
    mesh=_MESH,
    scratch_types=[
        pltpu.VMEM((K0, CH), jnp.int32),
        pltpu.VMEM((1, CH), jnp.int32),
        pltpu.VMEM((1, CH), jnp.int32),
        pltpu.VMEM((CH, 128), _F32),
        pltpu.VMEM((CH, 128), _F32),
        pltpu.VMEM_SHARED((NA, 128), _F32),
        pltpu.SemaphoreType.DMA,
        pltpu.SemaphoreType.DMA,
        pltpu.SemaphoreType.DMA,
        pltpu.SemaphoreType.DMA,
    ],
)


def _agg2_body(tab_a, tab_b, srcs_hbm, dsts_hbm, z_hbm, out_a, out_b,
               dst_v, x0, x1, b0, b1, acc,
               i0, i1, g0, g1):
    sidx = (x0, x1)
    bufs = (b0, b1)
    isems = (i0, i1)
    gsems = (g0, g1)
    c = lax.axis_index("c")
    s = lax.axis_index("s")
    base = jnp.where(c == 1, s * K1, NS * K1 + s * K0)
    kc = jnp.where(c == 1, K1, K0)
    pltpu.sync_copy(dsts_hbm.at[pl.ds(base, K0)], dst_v)

    for tab_hbm, out_hbm in ((tab_a, out_a), (tab_b, out_b)):
        pltpu.sync_copy(z_hbm, acc.at[pl.ds(s * STRIPE, STRIPE)])
        plsc.subcore_barrier()

        for i in range(2):
            pltpu.async_copy(srcs_hbm.at[pl.ds(base + i, 1)],
                             sidx[i], isems[i])
        for i in range(2):
            pltpu.make_async_copy(srcs_hbm.at[pl.ds(base + i, 1)],
                                  sidx[i], isems[i]).wait()
            pltpu.async_copy(tab_hbm.at[sidx[i].at[0]], bufs[i], gsems[i])

        def body(t, carry):
            j = 2 * t
            for i in range(2):
                pltpu.make_async_copy(tab_hbm.at[sidx[i].at[0]],
                                      bufs[i], gsems[i]).wait()

                @pl.when(j + i + 2 < kc)
                def _():
                    pltpu.async_copy(srcs_hbm.at[pl.ds(base + j + i + 2, 1)],
                                     sidx[i], isems[i])

                pltpu.sync_copy(bufs[i], acc.at[dst_v.at[j + i]], add=True)

                @pl.when(j + i + 2 < kc)
                def _():
                    pltpu.make_async_copy(
                        srcs_hbm.at[pl.ds(base + j + i + 2, 1)],
                        sidx[i], isems[i]).wait()
                    pltpu.async_copy(tab_hbm.at[sidx[i].at[0]],
                                     bufs[i], gsems[i])
            return carry

        lax.fori_loop(0, kc // 2, body, 0)
        plsc.subcore_barrier()
        pltpu.sync_copy(acc.at[pl.ds(s * STRIPE, STRIPE)],
                        out_hbm.at[c, pl.ds(s * STRIPE, STRIPE)])


_agg2 = pl.kernel(
    _agg2_body,
    out_type=[jax.ShapeDtypeStruct((NC, NA, 128), _F32)] * 2,
    mesh=_MESH,
    scratch_types=[
        pltpu.VMEM((K0, CH), jnp.int32),
        pltpu.VMEM((1, CH), jnp.int32),
        pltpu.VMEM((1, CH), jnp.int32),
        pltpu.VMEM((CH, 128), _F32),
        pltpu.VMEM((CH, 128), _F32),
        pltpu.VMEM_SHARED((NA, 128), _F32),
        pltpu.SemaphoreType.DMA,
        pltpu.SemaphoreType.DMA,
        pltpu.SemaphoreType.DMA,
        pltpu.SemaphoreType.DMA,
    ],
)


def _deg_body(dsts_hbm, ones_hbm, z_hbm, out_hbm, dst_v, ones_v, acc):
    c = lax.axis_index("c")
    s = lax.axis_index("s")
    wid = s * NC + c
    pltpu.sync_copy(z_hbm, acc.at[pl.ds(s * STRIPE, STRIPE)])
    pltpu.sync_copy(ones_hbm, ones_v)
    pltpu.sync_copy(dsts_hbm.at[pl.ds(wid * KD, KD)], dst_v)
    plsc.subcore_barrier()

    def chunk(j, carry):
        pltpu.sync_copy(ones_v, acc.at[dst_v.at[j]], add=True)
        return carry

    lax.fori_loop(0, KD, chunk, 0)
    plsc.subcore_barrier()
    pltpu.sync_copy(acc.at[pl.ds(s * STRIPE, STRIPE)],
                    out_hbm.at[c, pl.ds(s * STRIPE, STRIPE)])


_deg = pl.kernel(
    _deg_body,
    out_type=jax.ShapeDtypeStruct((NC, NA, 128), _F32),
    mesh=_MESH,
    scratch_types=[
        pltpu.VMEM((KD, CH), jnp.int32),
        pltpu.VMEM((CH, 128), _F32),
        pltpu.VMEM_SHARED((NA, 128), _F32),
    ],
)


_R = 1000


def _tc1_body(degp, x, dinv_b, xs):
    deg = degp[0] + degp[1] + 1.0
    db = lax.rsqrt(deg)
    dinv_b[...] = db
    xs[...] = x[...] * db


def _tc2_body(p1, x, dinv, w1, b1, h1, hs1a, hs1b):
    db = dinv[...]
    z = db * (p1[0] + p1[1]) + db * db * x[...]
    h = jnp.maximum(
        jnp.dot(z, w1[...], preferred_element_type=_F32,
                precision=lax.Precision.HIGHEST) + b1[...], 0.0)
    h1[...] = h
    d256 = jnp.broadcast_to(db[:, 0:1], (_R, 256))
    hs = h * d256
    hs1a[...] = hs[:, :128]
    hs1b[...] = hs[:, 128:]


def _tc3_body(p2a, p2b, h1, dinv, w2, b2, w3, g, gs):
    db = dinv[...]
    d256 = jnp.broadcast_to(db[:, 0:1], (_R, 256))
    agg = jnp.concatenate([p2a[0] + p2a[1], p2b[0] + p2b[1]], axis=1)
    z2 = d256 * agg + d256 * d256 * h1[...]
    h2 = jnp.maximum(
        jnp.dot(z2, w2[...], preferred_element_type=_F32,
                precision=lax.Precision.HIGHEST) + b2[...], 0.0)
    gg = jnp.dot(h2, w3[...], preferred_element_type=_F32,
                 precision=lax.Precision.HIGHEST)
    g[...] = gg
    gs[...] = gg * db


def _tc4_body(p3, g, dinv, b3, out):
    db = dinv[...]
    out[...] = db * (p3[0] + p3[1]) + db * db * g[...] + b3[...]


def _row_spec(d):
    return pl.BlockSpec((_R, d), lambda i: (i, 0))


def _part_spec(d):
    return pl.BlockSpec((NC, _R, d), lambda i: (0, i, 0))


def _full_spec(a, b):
    return pl.BlockSpec((a, b), lambda i: (0, 0))


_GRID = N // _R

_tc1 = pl.pallas_call(
    _tc1_body,
    grid=(_GRID,),
    in_specs=[_part_spec(128), _row_spec(128)],
    out_specs=[_row_spec(128), _row_spec(128)],
    out_shape=[jax.ShapeDtypeStruct((N, 128), _F32)] * 2,
)

_tc2 = pl.pallas_call(
    _tc2_body,
    grid=(_GRID,),
    in_specs=[_part_spec(128), _row_spec(128), _row_spec(128),
              _full_spec(128, 256), _full_spec(1, 256)],
    out_specs=[_row_spec(256), _row_spec(128), _row_spec(128)],
    out_shape=[jax.ShapeDtypeStruct((N, 256), _F32),
               jax.ShapeDtypeStruct((N, 128), _F32),
               jax.ShapeDtypeStruct((N, 128), _F32)],
)

_tc3 = pl.pallas_call(
    _tc3_body,
    grid=(_GRID,),
    in_specs=[_part_spec(128), _part_spec(128), _row_spec(256), _row_spec(128),
              _full_spec(256, 256), _full_spec(1, 256), _full_spec(256, 128)],
    out_specs=[_row_spec(128), _row_spec(128)],
    out_shape=[jax.ShapeDtypeStruct((N, 128), _F32)] * 2,
)

_tc4 = pl.pallas_call(
    _tc4_body,
    grid=(_GRID,),
    in_specs=[_part_spec(128), _row_spec(128), _row_spec(128),
              _full_spec(1, 128)],
    out_specs=_row_spec(128),
    out_shape=jax.ShapeDtypeStruct((N, 128), _F32),
)


@jax.jit
def kernel(x, edge_index, W1, b1, W2, b2, W3, b3):
    pad = EP - E
    pad_src = (jnp.arange(pad, dtype=jnp.int32) % 128)
    pad_dst = N + (jnp.arange(pad, dtype=jnp.int32) % (NA - N))
    src = jnp.concatenate([edge_index[0], pad_src]).reshape(TOT, CH)
    dst = jnp.concatenate([edge_index[1], pad_dst]).reshape(TOT, CH)
    z128 = jnp.zeros((STRIPE, 128), _F32)
    ones128 = jnp.ones((CH, 128), _F32)

    degp = _deg(dst, ones128, z128)
    dinv_b, xs = _tc1(degp, x)
    p1 = _agg(xs, src, dst, z128)
    h1, hs1a, hs1b = _tc2(p1, x, dinv_b, W1, b1.reshape(1, 256))
    p2a, p2b = _agg2(hs1a, hs1b, src, dst, z128)
    g, gs = _tc3(p2a, p2b, h1, dinv_b, W2, b2.reshape(1, 256), W3)
    p3 = _agg(gs, src, dst, z128)
    return _tc4(p3, g, dinv_b, b3.reshape(1, 128))
